# baseline (speedup 1.0000x reference)
import functools

import jax
import jax.numpy as jnp
from jax.experimental import pallas as pl

B = 4096
IN_DIM = 3072
Z = 4096
K = 256
L = 16
H = 512
TILE = 128

_PREC = jax.lax.Precision.DEFAULT
_EXACT = jax.lax.Precision.HIGHEST


def _body(x_ref, W1_ref, b1_ref, W2_ref, b2_ref, w_ref, wT_ref, wsq_ref,
          W3_ref, b3_ref, W4_ref, b4_ref, z_ref, emb_ref, recon_ref):
    x = x_ref[...]
    h = jax.nn.relu(
        jnp.dot(x, W1_ref[...], precision=_PREC, preferred_element_type=jnp.float32)
        + b1_ref[...])
    zz = (jnp.dot(h, W2_ref[...], precision=_PREC, preferred_element_type=jnp.float32)
          + b2_ref[...])
    z_ref[...] = zz[:, :Z]
    zt = zz[:, Z:]

    w = w_ref[...]
    wT = wT_ref[...]
    wsq = wsq_ref[...]
    iota = jax.lax.broadcasted_iota(jnp.int32, (TILE, K), 1)
    acc = b3_ref[...] * jnp.ones((TILE, 1), jnp.float32)
    for l in range(L):
        xl = zt[:, l * K:(l + 1) * K]
        t = jnp.dot(xl, w, precision=_PREC, preferred_element_type=jnp.float32)
        xsq = jnp.sum(xl * xl, axis=1, keepdims=True)
        d = xsq - 2.0 * t + wsq
        dmin = jnp.min(d, axis=1, keepdims=True)
        idx = jnp.min(jnp.where(d == dmin, iota, K), axis=1, keepdims=True)
        onehot = (iota == idx).astype(jnp.float32)
        E = jnp.dot(onehot, wT, precision=_EXACT, preferred_element_type=jnp.float32)
        emb_ref[:, l * K:(l + 1) * K] = E
        acc = acc + jnp.dot(E, W3_ref[l * K:(l + 1) * K, :], precision=_PREC,
                            preferred_element_type=jnp.float32)
    h2 = jax.nn.relu(acc)
    recon_ref[...] = jax.nn.sigmoid(
        jnp.dot(h2, W4_ref[...], precision=_PREC, preferred_element_type=jnp.float32)
        + b4_ref[...])


@functools.partial(jax.jit, static_argnums=())
def kernel(x, W1, b1, W2, b2, emb_weight, W3, b3, W4, b4):
    W1p = jnp.pad(W1, ((0, 0), (0, H - 400)))
    b1p = jnp.pad(b1, (0, H - 400)).reshape(1, H)
    W2r = jnp.pad(W2, ((0, H - 400), (0, 0)))
    W2p = jnp.concatenate(
        [W2r, W2r.reshape(H, K, L).transpose(0, 2, 1).reshape(H, Z)], axis=1)
    b2p = jnp.concatenate([b2, b2.reshape(K, L).T.reshape(Z)]).reshape(1, 2 * Z)
    w = emb_weight
    wT = emb_weight.T
    wsq = jnp.sum(w * w, axis=0).reshape(1, K)
    W3p = jnp.pad(W3, ((0, 0), (0, H - 400)))
    W3r = W3p.reshape(K, L, H).transpose(1, 0, 2).reshape(Z, H)
    b3p = jnp.pad(b3, (0, H - 400)).reshape(1, H)
    W4p = jnp.pad(W4, ((0, H - 400), (0, 0)))
    b4r = b4.reshape(1, IN_DIM)

    n_tiles = B // TILE
    full = lambda shape: pl.BlockSpec(shape, lambda i: (0, 0))
    z, emb_flat, recon = pl.pallas_call(
        _body,
        grid=(n_tiles,),
        in_specs=[
            pl.BlockSpec((TILE, IN_DIM), lambda i: (i, 0)),
            full((IN_DIM, H)), full((1, H)),
            full((H, 2 * Z)), full((1, 2 * Z)),
            full((K, K)), full((K, K)), full((1, K)),
            full((Z, H)), full((1, H)),
            full((H, IN_DIM)), full((1, IN_DIM)),
        ],
        out_specs=[
            pl.BlockSpec((TILE, Z), lambda i: (i, 0)),
            pl.BlockSpec((TILE, Z), lambda i: (i, 0)),
            pl.BlockSpec((TILE, IN_DIM), lambda i: (i, 0)),
        ],
        out_shape=[
            jax.ShapeDtypeStruct((B, Z), jnp.float32),
            jax.ShapeDtypeStruct((B, Z), jnp.float32),
            jax.ShapeDtypeStruct((B, IN_DIM), jnp.float32),
        ],
    )(x, W1p, b1p, W2p, b2p, w, wT, wsq, W3r, b3p, W4p, b4r)

    z_e = z.reshape(B, K, L)
    emb = emb_flat.reshape(B, L, K).transpose(0, 2, 1)
    return (recon, z_e, emb)


# speedup vs baseline: 1.5466x; 1.5466x over previous
import functools

import jax
import jax.numpy as jnp
from jax.experimental import pallas as pl

B = 4096
IN_DIM = 3072
Z = 4096
K = 256
L = 16
H = 512
TILE = 128

_PREC = jax.lax.Precision.DEFAULT
_EXACT = jax.lax.Precision.HIGHEST


def _dot(a, b, prec=_PREC):
    return jnp.dot(a, b, precision=prec, preferred_element_type=jnp.float32)


def _body(x_ref, W1_ref, b1_ref, W2_ref, b2_ref, w_ref, wT_ref, wsq_ref,
          G_ref, b3_ref, W4_ref, b4_ref, zt_ref, emb_ref, recon_ref):
    x = x_ref[...]
    h = jax.nn.relu(_dot(x, W1_ref[...]) + b1_ref[...])
    zt = _dot(h, W2_ref[...]) + b2_ref[...]
    zt_ref[...] = zt

    w = w_ref[...]
    wsq = wsq_ref[...]
    iota = jax.lax.broadcasted_iota(jnp.int32, (TILE, K), 1)

    ts = [_dot(zt[:, l * K:(l + 1) * K], w) for l in range(L)]
    os = []
    for l in range(L):
        xl = zt[:, l * K:(l + 1) * K]
        xsq = jnp.sum(xl * xl, axis=1, keepdims=True)
        d = xsq - 2.0 * ts[l] + wsq
        dmin = jnp.min(d, axis=1, keepdims=True)
        idx = jnp.min(jnp.where(d == dmin, iota, K), axis=1, keepdims=True)
        os.append((iota == idx).astype(jnp.float32))
    O_all = jnp.concatenate(os, axis=1)
    acc = _dot(O_all, G_ref[...]) + b3_ref[...]
    wT = wT_ref[...]
    for l in range(L):
        emb_ref[:, l * K:(l + 1) * K] = _dot(os[l], wT, prec=_EXACT)
    h2 = jax.nn.relu(acc)
    recon_ref[...] = jax.nn.sigmoid(_dot(h2, W4_ref[...]) + b4_ref[...])


@functools.partial(jax.jit, static_argnums=())
def kernel(x, W1, b1, W2, b2, emb_weight, W3, b3, W4, b4):
    W1p = jnp.pad(W1, ((0, 0), (0, H - 400)))
    b1p = jnp.pad(b1, (0, H - 400)).reshape(1, H)
    W2r = jnp.pad(W2, ((0, H - 400), (0, 0)))
    W2p = W2r.reshape(H, K, L).transpose(0, 2, 1).reshape(H, Z)
    b2p = b2.reshape(K, L).T.reshape(1, Z)
    w = emb_weight
    wT = emb_weight.T
    wsq = jnp.sum(w * w, axis=0).reshape(1, K)
    W3p = jnp.pad(W3, ((0, 0), (0, H - 400)))
    W3r3 = W3p.reshape(K, L, H)
    G = jnp.einsum('dk,dlh->lkh', w, W3r3,
                   precision=_PREC,
                   preferred_element_type=jnp.float32).reshape(Z, H)
    b3p = jnp.pad(b3, (0, H - 400)).reshape(1, H)
    W4p = jnp.pad(W4, ((0, H - 400), (0, 0)))
    b4r = b4.reshape(1, IN_DIM)

    n_tiles = B // TILE
    full = lambda shape: pl.BlockSpec(shape, lambda i: (0, 0))
    zt, emb_flat, recon = pl.pallas_call(
        _body,
        grid=(n_tiles,),
        in_specs=[
            pl.BlockSpec((TILE, IN_DIM), lambda i: (i, 0)),
            full((IN_DIM, H)), full((1, H)),
            full((H, Z)), full((1, Z)),
            full((K, K)), full((K, K)), full((1, K)),
            full((Z, H)), full((1, H)),
            full((H, IN_DIM)), full((1, IN_DIM)),
        ],
        out_specs=[
            pl.BlockSpec((TILE, Z), lambda i: (i, 0)),
            pl.BlockSpec((TILE, Z), lambda i: (i, 0)),
            pl.BlockSpec((TILE, IN_DIM), lambda i: (i, 0)),
        ],
        out_shape=[
            jax.ShapeDtypeStruct((B, Z), jnp.float32),
            jax.ShapeDtypeStruct((B, Z), jnp.float32),
            jax.ShapeDtypeStruct((B, IN_DIM), jnp.float32),
        ],
    )(x, W1p, b1p, W2p, b2p, w, wT, wsq, G, b3p, W4p, b4r)

    z_e = zt.reshape(B, L, K).transpose(0, 2, 1)
    emb = emb_flat.reshape(B, L, K).transpose(0, 2, 1)
    return (recon, z_e, emb)
